# Initial kernel scaffold; baseline (speedup 1.0000x reference)
import jax
import jax.numpy as jnp
from jax.experimental import pallas as pl


def kernel(z, embedding):
    raise NotImplementedError("write your pallas kernel here")


# speedup vs baseline: 1.0463x; 1.0463x over previous
import functools

import jax
import jax.numpy as jnp
from jax import lax
from jax.experimental import pallas as pl
from jax.experimental.pallas import tpu as pltpu
from jax.experimental.pallas import tpu_sc as plsc

N_EMB = 8192
DIM = 32
N_TOK = 32 * 1024


TM = 256


def _argmin_body(z_ref, emb_ref, idx_ref):
    z_blk = z_ref[...]
    emb = emb_ref[...]
    d = lax.dot_general(z_blk.astype(jnp.bfloat16), emb.astype(jnp.bfloat16),
                        (((1,), (1,)), ((), ())),
                        preferred_element_type=jnp.float32)
    z2 = jnp.sum(z_blk * z_blk, axis=1, keepdims=True)
    e2 = jnp.sum(emb * emb, axis=1)[None, :]
    dist = (z2 - 2.0 * d) + e2
    m = jnp.min(dist, axis=1, keepdims=True)
    iota = lax.broadcasted_iota(jnp.int32, dist.shape, 1)
    idx_ref[...] = jnp.min(jnp.where(dist == m, iota, jnp.int32(N_EMB)),
                           axis=1)


def _argmin_call(z_flat, embedding, interpret=False):
    nb = N_TOK // TM
    return pl.pallas_call(
        _argmin_body,
        grid=(nb,),
        in_specs=[
            pl.BlockSpec((TM, DIM), lambda i: (i, 0)),
            pl.BlockSpec((N_EMB, DIM), lambda i: (0, 0)),
        ],
        out_specs=pl.BlockSpec((TM,), lambda i: (i,)),
        out_shape=jax.ShapeDtypeStruct((N_TOK,), jnp.int32),
        interpret=interpret,
    )(z_flat, embedding)


NW = 32
B_PER_W = N_TOK // NW
CHUNK = 128


@functools.cache
def _make_gather():
    @functools.partial(
        pl.kernel,
        out_type=jax.ShapeDtypeStruct((N_TOK, DIM), jnp.float32),
        mesh=plsc.VectorSubcoreMesh(core_axis_name="c", subcore_axis_name="s"),
        scratch_types=[
            pltpu.VMEM((CHUNK,), jnp.int32),
            pltpu.VMEM((B_PER_W, DIM), jnp.float32),
            pltpu.SemaphoreType.DMA,
        ],
        compiler_params=pltpu.CompilerParams(use_tc_tiling_on_sc=False),
    )
    def _gather_k(idx_hbm, table_hbm, out_hbm, idx_v, rows_v, sem):
        wid = lax.axis_index("s") * 2 + lax.axis_index("c")
        base = wid * B_PER_W
        for j in range(B_PER_W // CHUNK):
            pltpu.sync_copy(idx_hbm.at[pl.ds(base + j * CHUNK, CHUNK)], idx_v)
            pltpu.async_copy(table_hbm.at[idx_v],
                             rows_v.at[pl.ds(j * CHUNK, CHUNK)], sem).wait()
        pltpu.sync_copy(rows_v, out_hbm.at[pl.ds(base, B_PER_W)])

    return _gather_k


def kernel(z, embedding):
    z_flat = z.reshape(-1, DIM)
    idx_flat = _argmin_call(z_flat, embedding)
    zq = _make_gather()(idx_flat, embedding)
    return zq.reshape(z.shape), idx_flat.reshape(z.shape[0], -1)
